# baseline (speedup 1.0000x reference)
import jax
import jax.numpy as jnp
from jax import lax
from jax.experimental import pallas as pl
from jax.experimental.pallas import tpu as pltpu
from jax.experimental.pallas import tpu_sc as plsc

B = 16384 * 26
NC, NS = 2, 16
NW = NC * NS
BPW = B // NW
GROUP = 256
NGROUPS = BPW // GROUP
CHUNK = 128
NCHUNK = GROUP // CHUNK
VEC = 16


def _m12_body(c2a_ref, c2b_ref, c1_ref, outa_ref, outb_ref):
    c1 = c1_ref[...]
    outa_ref[...] = jnp.dot(c2a_ref[...], c1, preferred_element_type=jnp.float32)
    outb_ref[...] = jnp.dot(c2b_ref[...], c1, preferred_element_type=jnp.float32)


def _table_body(m_ref, w_ref, out_ref):
    res = jnp.dot(m_ref[...], w_ref[0], preferred_element_type=jnp.float32)
    out_ref[0:10000, :] = res[:, 0:128]
    out_ref[10000:20000, :] = res[:, 128:256]


def _build_table(c2a, c2b, c1flat, w):
    m12a, m12b = pl.pallas_call(
        _m12_body,
        out_shape=(
            jax.ShapeDtypeStruct((100, 6400), jnp.float32),
            jax.ShapeDtypeStruct((100, 6400), jnp.float32),
        ),
    )(c2a, c2b, c1flat)
    m12cat = jnp.concatenate(
        [m12a.reshape(10000, 64), m12b.reshape(10000, 64)], axis=1
    )
    t4 = pl.pallas_call(
        _table_body,
        grid=(13,),
        in_specs=[
            pl.BlockSpec((10000, 128), lambda i: (0, 0)),
            pl.BlockSpec((1, 128, 256), lambda i: (i, 0, 0)),
        ],
        out_specs=pl.BlockSpec((20000, 128), lambda i: (i, 0)),
        out_shape=jax.ShapeDtypeStruct((260000, 128), jnp.float32),
    )(m12cat, w)
    return t4


def _sc_body(x_ref, t_ref, out_ref, idx_h, row_v,
             buf_a, buf_b, stag_a, stag_b, sg_a, sg_b, so_a, so_b):
    wid = lax.axis_index("s") * NC + lax.axis_index("c")
    base = wid * BPW
    itbase = wid * 4
    iota = lax.iota(jnp.int32, 16)
    iota26 = iota * 26

    for h in range(2):
        pltpu.sync_copy(x_ref.at[pl.ds(base + h * 6656, 6656)], idx_h)

        def dig(t, carry):
            j = t >> 4
            ibl = t & 15
            lg = iota26 + (ibl * 416 + j)
            v = plsc.load_gather(idx_h, [lg])
            vf = v.astype(jnp.float32) + 0.5
            d0 = (vf * (1.0 / 10000.0)).astype(jnp.int32)
            rem = v - d0 * 10000
            d1 = ((rem.astype(jnp.float32) + 0.5) * (1.0 / 100.0)).astype(jnp.int32)
            d2 = rem - d1 * 100
            y = (d0 >> 2) * 40000 + (d2 * 100 + d1) * 4 + (d0 & 3)
            row_v[pl.ds(j * 512 + h * 256 + ibl * 16, 16)] = y
            return carry

        lax.fori_loop(0, 416, dig, 0)

    def fire(u, buf, sem):
        for c in range(2):
            pltpu.async_copy(
                t_ref.at[row_v.at[pl.ds(u * 256 + c * 128, 128)]],
                buf.at[pl.ds(c * 128, 128)],
                sem,
            )

    def gdrain(buf, sem):
        for c in range(2):
            pltpu.make_async_copy(
                t_ref.at[row_v.at[pl.ds(0, 128)]],
                buf.at[pl.ds(c * 128, 128)],
                sem,
            ).wait()

    def select(buf, stag):
        for itl in range(2):
            for ib in range(8):
                rowv = iota + (itl * 128 + ib * 16)
                for kt in range(4):
                    for km in range(8):
                        col = jnp.full((16,), kt * 8 + km, jnp.int32)
                        val = plsc.load_gather(buf, [rowv, col])
                        stag[kt, itl, km, pl.ds(ib * 16, 16)] = val

    def puts(j, h, stag, sem):
        for kt in range(4):
            pltpu.async_copy(
                stag.at[kt], out_ref.at[j, kt, pl.ds(itbase + h * 2, 2)], sem
            )

    def odrain(stag, sem):
        for kt in range(4):
            pltpu.make_async_copy(
                stag.at[kt], out_ref.at[0, kt, pl.ds(0, 2)], sem
            ).wait()

    fire(0, buf_a, sg_a)
    fire(1, buf_b, sg_b)

    def unit_pair(j, carry):
        gdrain(buf_a, sg_a)

        @pl.when(j >= 1)
        def _():
            odrain(stag_a, so_a)

        select(buf_a, stag_a)
        puts(j, 0, stag_a, so_a)
        fire(2 * j + 2, buf_a, sg_a)

        gdrain(buf_b, sg_b)

        @pl.when(j >= 1)
        def _():
            odrain(stag_b, so_b)

        select(buf_b, stag_b)
        puts(j, 1, stag_b, so_b)
        fire(2 * j + 3, buf_b, sg_b)
        return carry

    lax.fori_loop(0, 25, unit_pair, 0)
    gdrain(buf_a, sg_a)
    odrain(stag_a, so_a)
    select(buf_a, stag_a)
    puts(25, 0, stag_a, so_a)
    gdrain(buf_b, sg_b)
    odrain(stag_b, so_b)
    select(buf_b, stag_b)
    puts(25, 1, stag_b, so_b)
    odrain(stag_a, so_a)
    odrain(stag_b, so_b)


def _make_sc_lookup():
    return pl.kernel(
        _sc_body,
        out_type=jax.ShapeDtypeStruct((26, 4, 128, 8, 128), jnp.float32),
        compiler_params=pltpu.CompilerParams(
            use_tc_tiling_on_sc=False, needs_layout_passes=False
        ),
        mesh=plsc.VectorSubcoreMesh(
            core_axis_name="c", subcore_axis_name="s",
            num_cores=NC, num_subcores=NS,
        ),
        scratch_types=[
            pltpu.VMEM((6656,), jnp.int32),
            pltpu.VMEM((BPW,), jnp.int32),
            pltpu.VMEM((256, 32), jnp.float32),
            pltpu.VMEM((256, 32), jnp.float32),
            pltpu.VMEM((4, 2, 8, 128), jnp.float32),
            pltpu.VMEM((4, 2, 8, 128), jnp.float32),
            pltpu.SemaphoreType.DMA,
            pltpu.SemaphoreType.DMA,
            pltpu.SemaphoreType.DMA,
            pltpu.SemaphoreType.DMA,
        ],
    )


def kernel(x, core0, core1, core2):
    xf = x.reshape(-1)
    g0 = core0[0]
    c2 = core2[..., 0]
    c2a = c2[:, :, 0].T
    c2b = c2[:, :, 1].T
    c1flat = jnp.transpose(core1, (3, 1, 0, 2)).reshape(16, 6400)
    g0p = jnp.concatenate(
        [g0, jnp.zeros((4, 4, 16), jnp.float32)], axis=0
    ).reshape(13, 8, 4, 16)
    w = jnp.einsum(
        "pdar,xc,ye->pyrxdace",
        g0p,
        jnp.eye(4, dtype=jnp.float32),
        jnp.eye(2, dtype=jnp.float32),
    ).reshape(13, 128, 256)

    t4 = _build_table(c2a, c2b, c1flat, w)
    t = t4.reshape(1040000, 32)
    out5 = _make_sc_lookup()(xf, t)
    return jnp.transpose(out5, (2, 4, 0, 1, 3)).reshape(x.shape + (32,))


# speedup vs baseline: 2.4458x; 2.4458x over previous
import jax
import jax.numpy as jnp
from jax import lax
from jax.experimental import pallas as pl
from jax.experimental.pallas import tpu as pltpu
from jax.experimental.pallas import tpu_sc as plsc

B = 16384 * 26
NC, NS = 2, 16
NW = NC * NS
BPW = B // NW
GROUP = 256
NGROUPS = BPW // GROUP
CHUNK = 128
NCHUNK = GROUP // CHUNK
VEC = 16


def _m12_body(c2a_ref, c2b_ref, c1_ref, outa_ref, outb_ref):
    c1 = c1_ref[...]
    outa_ref[...] = jnp.dot(c2a_ref[...], c1, preferred_element_type=jnp.float32)
    outb_ref[...] = jnp.dot(c2b_ref[...], c1, preferred_element_type=jnp.float32)


def _table_body(m_ref, w_ref, out_ref):
    res = jnp.dot(m_ref[...], w_ref[0], preferred_element_type=jnp.float32)
    out_ref[0:10000, :] = res[:, 0:128]
    out_ref[10000:20000, :] = res[:, 128:256]


def _build_table(c2a, c2b, c1flat, w):
    m12a, m12b = pl.pallas_call(
        _m12_body,
        out_shape=(
            jax.ShapeDtypeStruct((100, 6400), jnp.float32),
            jax.ShapeDtypeStruct((100, 6400), jnp.float32),
        ),
    )(c2a, c2b, c1flat)
    m12cat = jnp.concatenate(
        [m12a.reshape(10000, 64), m12b.reshape(10000, 64)], axis=1
    )
    t4 = pl.pallas_call(
        _table_body,
        grid=(13,),
        in_specs=[
            pl.BlockSpec((10000, 128), lambda i: (0, 0)),
            pl.BlockSpec((1, 128, 256), lambda i: (i, 0, 0)),
        ],
        out_specs=pl.BlockSpec((20000, 128), lambda i: (i, 0)),
        out_shape=jax.ShapeDtypeStruct((260000, 128), jnp.float32),
    )(m12cat, w)
    return t4


def _sc_body(x_ref, t_ref, out_ref, idx_h, row_v,
             buf_a, buf_b, stag_a, stag_b, sg_a, sg_b, so_a, so_b):
    wid = lax.axis_index("s") * NC + lax.axis_index("c")
    base = wid * BPW
    itbase = wid * 4
    iota = lax.iota(jnp.int32, 16)
    iota26 = iota * 26

    for h in range(2):
        pltpu.sync_copy(x_ref.at[pl.ds(base + h * 6656, 6656)], idx_h)

        def dig(t, carry):
            j = t >> 4
            ibl = t & 15
            lg = iota26 + (ibl * 416 + j)
            v = plsc.load_gather(idx_h, [lg])
            vf = v.astype(jnp.float32) + 0.5
            d0 = (vf * (1.0 / 10000.0)).astype(jnp.int32)
            rem = v - d0 * 10000
            d1 = ((rem.astype(jnp.float32) + 0.5) * (1.0 / 100.0)).astype(jnp.int32)
            d2 = rem - d1 * 100
            y = (d0 >> 2) * 40000 + (d2 * 100 + d1) * 4 + (d0 & 3)
            row_v[pl.ds(j * 512 + h * 256 + ibl * 16, 16)] = y
            return carry

        lax.fori_loop(0, 416, dig, 0)

    def fire(u, buf, sem):
        for c in range(2):
            pltpu.async_copy(
                t_ref.at[row_v.at[pl.ds(u * 256 + c * 128, 128)]],
                buf.at[pl.ds(c * 128, 128)],
                sem,
            )

    def gdrain(buf, sem):
        for c in range(2):
            pltpu.make_async_copy(
                t_ref.at[row_v.at[pl.ds(0, 128)]],
                buf.at[pl.ds(c * 128, 128)],
                sem,
            ).wait()

    ktv0 = iota >> 3
    ktv1 = ktv0 + 2
    kmv = iota & 7

    def select(buf, stag):
        def one(dl, carry):
            itl = dl >> 7
            im = dl & 127
            itlv = jnp.full((16,), itl, jnp.int32)
            imv = jnp.full((16,), im, jnp.int32)
            v0 = buf[dl, pl.ds(0, 16)]
            v1 = buf[dl, pl.ds(16, 16)]
            plsc.store_scatter(stag, [ktv0, itlv, kmv, imv], v0)
            plsc.store_scatter(stag, [ktv1, itlv, kmv, imv], v1)
            return carry

        lax.fori_loop(0, 256, one, 0)

    def puts(j, h, stag, sem):
        for kt in range(4):
            pltpu.async_copy(
                stag.at[kt, :, :, pl.ds(0, 128)],
                out_ref.at[j, kt, pl.ds(itbase + h * 2, 2)],
                sem,
            )

    def odrain(stag, sem):
        for kt in range(4):
            pltpu.make_async_copy(
                stag.at[kt, :, :, pl.ds(0, 128)],
                out_ref.at[0, kt, pl.ds(0, 2)],
                sem,
            ).wait()

    fire(0, buf_a, sg_a)
    fire(1, buf_b, sg_b)

    def unit_pair(j, carry):
        gdrain(buf_a, sg_a)

        @pl.when(j >= 1)
        def _():
            odrain(stag_a, so_a)

        select(buf_a, stag_a)
        puts(j, 0, stag_a, so_a)
        fire(2 * j + 2, buf_a, sg_a)

        gdrain(buf_b, sg_b)

        @pl.when(j >= 1)
        def _():
            odrain(stag_b, so_b)

        select(buf_b, stag_b)
        puts(j, 1, stag_b, so_b)
        fire(2 * j + 3, buf_b, sg_b)
        return carry

    lax.fori_loop(0, 25, unit_pair, 0)
    gdrain(buf_a, sg_a)
    odrain(stag_a, so_a)
    select(buf_a, stag_a)
    puts(25, 0, stag_a, so_a)
    gdrain(buf_b, sg_b)
    odrain(stag_b, so_b)
    select(buf_b, stag_b)
    puts(25, 1, stag_b, so_b)
    odrain(stag_a, so_a)
    odrain(stag_b, so_b)


def _make_sc_lookup():
    return pl.kernel(
        _sc_body,
        out_type=jax.ShapeDtypeStruct((26, 4, 128, 8, 128), jnp.float32),
        compiler_params=pltpu.CompilerParams(
            use_tc_tiling_on_sc=False, needs_layout_passes=False
        ),
        mesh=plsc.VectorSubcoreMesh(
            core_axis_name="c", subcore_axis_name="s",
            num_cores=NC, num_subcores=NS,
        ),
        scratch_types=[
            pltpu.VMEM((6656,), jnp.int32),
            pltpu.VMEM((BPW,), jnp.int32),
            pltpu.VMEM((256, 32), jnp.float32),
            pltpu.VMEM((256, 32), jnp.float32),
            pltpu.VMEM((4, 2, 8, 130), jnp.float32),
            pltpu.VMEM((4, 2, 8, 130), jnp.float32),
            pltpu.SemaphoreType.DMA,
            pltpu.SemaphoreType.DMA,
            pltpu.SemaphoreType.DMA,
            pltpu.SemaphoreType.DMA,
        ],
    )


def kernel(x, core0, core1, core2):
    xf = x.reshape(-1)
    g0 = core0[0]
    c2 = core2[..., 0]
    c2a = c2[:, :, 0].T
    c2b = c2[:, :, 1].T
    c1flat = jnp.transpose(core1, (3, 1, 0, 2)).reshape(16, 6400)
    g0p = jnp.concatenate(
        [g0, jnp.zeros((4, 4, 16), jnp.float32)], axis=0
    ).reshape(13, 8, 4, 16)
    w = jnp.einsum(
        "pdar,xc,ye->pyrxdace",
        g0p,
        jnp.eye(4, dtype=jnp.float32),
        jnp.eye(2, dtype=jnp.float32),
    ).reshape(13, 128, 256)

    t4 = _build_table(c2a, c2b, c1flat, w)
    t = t4.reshape(1040000, 32)
    out5 = _make_sc_lookup()(xf, t)
    return jnp.transpose(out5, (2, 4, 0, 1, 3)).reshape(x.shape + (32,))
